# baseline (speedup 1.0000x reference)
import jax
import jax.numpy as jnp
from jax import lax
from jax.experimental import pallas as pl
from jax.experimental.pallas import tpu as pltpu
from jax.experimental.pallas import tpu_sc as plsc

B = 1024
L = 200
D = 64
N = B * L
NC = 2
NS = 16
NW = NC * NS
TOK_PER_W = N // NW
CHUNK = 640
GG = 128
NGG = CHUNK // GG
NCHUNK = TOK_PER_W // CHUNK
NGRP = CHUNK // 16
EPS = 1e-12


def _tec_body(idx2d, seg2d, emb, segtab, pos, gb, out,
              idx_v, seg_v, rows, comb, pos_v, segt_v, gb_v, sem):
    wid = lax.axis_index("s") * NC + lax.axis_index("c")

    pltpu.sync_copy(pos.at[pl.ds(0, L)], pos_v)
    pltpu.sync_copy(segtab, segt_v)
    pltpu.sync_copy(gb, gb_v)

    def build_comb(l, c):
        for j in range(4):
            p = pos_v[l, pl.ds(j * 16, 16)]
            comb[pl.ds(l * 128 + j * 16, 16)] = p + segt_v[0, pl.ds(j * 16, 16)]
            comb[pl.ds(l * 128 + 64 + j * 16, 16)] = p + segt_v[1, pl.ds(j * 16, 16)]
        return c
    lax.fori_loop(0, L, build_comb, 0)

    iota16 = lax.iota(jnp.int32, 16)
    zeros = jnp.zeros((16,), jnp.float32)

    def chunk_body(c, carry):
        t0 = wid * TOK_PER_W + c * CHUNK
        pltpu.sync_copy(idx2d.at[pl.ds(t0, CHUNK)], idx_v)
        pltpu.sync_copy(seg2d.at[pl.ds(t0, CHUNK)], seg_v)
        descs = [
            pltpu.async_copy(emb.at[idx_v.at[pl.ds(j * GG, GG)]],
                             rows.at[pl.ds(j * GG, GG)], sem)
            for j in range(NGG)
        ]
        for dsc in descs:
            dsc.wait()

        def grp_body(g, gcarry):
            tl = iota16 + g * 16
            lvec = lax.rem(tl + t0, L)
            sv = seg_v[pl.ds(g * 16, 16)]
            cbase = (lvec * 2 + sv) * D
            tbase = tl * D

            def p1(d, acc):
                s_a, q_a = acc
                dv = jnp.broadcast_to(d, (16,)).astype(jnp.int32)
                v = (plsc.load_gather(rows, [tl, dv])
                     + plsc.load_gather(comb, [cbase + dv]))
                comb[pl.ds(25600 + g * 1024 + d * 16, 16)] = v
                return (s_a + v, q_a + v * v)
            s_a, q_a = lax.fori_loop(0, D, p1, (zeros, zeros), unroll=4)

            mean = s_a * (1.0 / D)
            var = q_a * (1.0 / D) - mean * mean
            xv = var + EPS
            iv = plsc.bitcast(
                jnp.full((16,), 0x5F3759DF, jnp.int32)
                - lax.shift_right_logical(plsc.bitcast(xv, jnp.int32), 1),
                jnp.float32)
            for _ in range(2):
                iv = iv * (1.5 - 0.5 * xv * iv * iv)

            def p2(d, acc):
                dv = jnp.broadcast_to(d, (16,)).astype(jnp.int32)
                v = comb[pl.ds(25600 + g * 1024 + d * 16, 16)]
                gv = plsc.load_gather(gb_v, [dv])
                bv = plsc.load_gather(gb_v, [dv + D])
                o = (v - mean) * iv * gv + bv
                plsc.store_scatter(rows, [tl, dv], o)
                return acc
            lax.fori_loop(0, D, p2, 0, unroll=4)
            return gcarry
        lax.fori_loop(0, NGRP, grp_body, 0)
        pltpu.sync_copy(rows, out.at[pl.ds(t0, CHUNK)])
        return carry
    lax.fori_loop(0, NCHUNK, chunk_body, 0)


def kernel(inputs, segments, emb_table, seg_table, pos_emb, ln_gamma, ln_beta):
    idx2d = inputs.reshape(N)
    seg2d = segments.reshape(N)
    gb = jnp.concatenate([ln_gamma, ln_beta])
    mesh = plsc.VectorSubcoreMesh(core_axis_name="c", subcore_axis_name="s")
    run = pl.kernel(
        _tec_body,
        out_type=jax.ShapeDtypeStruct((N, D), jnp.float32),
        mesh=mesh,
        compiler_params=pltpu.CompilerParams(
            needs_layout_passes=False, use_tc_tiling_on_sc=False),
        scratch_types=[
            pltpu.VMEM((CHUNK,), jnp.int32),
            pltpu.VMEM((CHUNK,), jnp.int32),
            pltpu.VMEM((CHUNK, D), jnp.float32),
            pltpu.VMEM((25600 + CHUNK * 64,), jnp.float32),
            pltpu.VMEM((L, D), jnp.float32),
            pltpu.VMEM((2, D), jnp.float32),
            pltpu.VMEM((2 * D,), jnp.float32),
            pltpu.SemaphoreType.DMA,
        ],
    )
    out = run(idx2d, seg2d, emb_table, seg_table, pos_emb, gb)
    return out.reshape(B, L, D)
